# baseline (speedup 1.0000x reference)
import jax
import jax.numpy as jnp
from jax import lax
from jax.experimental import pallas as pl
from jax.experimental.pallas import tpu as pltpu
from jax.experimental.pallas import tpu_sc as plsc

B = 16
M = 2048
F = 512
N = 16384

NC = 2
NS = 16
NW = NC * NS
TRG = 32
XTR = N // 8 * 4 * 8
DTR = B * M // 8 * 4 * 8

GC = 4
CR = GC * 8
CTR = GC * TRG
NCHUNK = B * M // CR
CPB = M // CR
CPT = NCHUNK // NW


def _iota16():
    return lax.broadcasted_iota(jnp.int32, (16,), 0)


def _build_idx(idxref, srcbase):
    for t0 in range(0, CTR, 16):
        tv = _iota16() + t0
        j = (tv >> 3) & 3
        sr = srcbase + ((tv >> 5) << 3) + (tv & 7)
        src = jnp.clip(sr, 0, N - 1)
        idxref[pl.ds(t0, 16)] = ((src >> 3) << 5) + (j << 3) + (src & 7)


def _sc_dense_kernel(x_hbm, params_hbm, out_hbm, gbuf, zbuf, pbuf, idx, sem):
    wid = lax.axis_index("s") * NC + lax.axis_index("c")

    pltpu.sync_copy(params_hbm.at[pl.ds(wid * CPT, CPT)], pbuf)

    @pl.loop(0, CTR)
    def _(i):
        @pl.loop(0, 128, step=16)
        def _(j):
            zbuf[i, pl.ds(j, 16)] = jnp.zeros((16,), jnp.float32)

    def chunk(i, _):
        row = pbuf[i]
        srcbase = row[0]
        vlen = row[1]
        dchunk = row[2]

        @pl.when(vlen == 0)
        def _():
            pltpu.async_copy(zbuf, out_hbm.at[pl.ds(dchunk, CTR)], sem)

        @pl.when(vlen > 0)
        def _():
            _build_idx(idx, srcbase)
            pltpu.sync_copy(x_hbm.at[idx], gbuf)

            @pl.when(vlen < CR)
            def _():
                def fix(dr, _):
                    base_tr = ((dr >> 3) << 5) + (dr & 7)
                    for j in range(4):
                        @pl.loop(0, 128, step=16)
                        def _(c, j=j):
                            gbuf[base_tr + (j << 3), pl.ds(c, 16)] = (
                                jnp.zeros((16,), jnp.float32)
                            )
                    return 0

                lax.fori_loop(vlen, CR, fix, 0)

            pltpu.sync_copy(gbuf, out_hbm.at[pl.ds(dchunk, CTR)])

        return 0

    lax.fori_loop(0, CPT, chunk, 0)

    def drain(i, _):
        row = pbuf[i]
        dchunk = row[2]

        @pl.when(row[1] == 0)
        def _():
            pltpu.make_async_copy(
                zbuf, out_hbm.at[pl.ds(dchunk, CTR)], sem
            ).wait()

        return 0

    lax.fori_loop(0, CPT, drain, 0)


def _sc_dense(x, params):
    x2 = (
        x.reshape(N // 8, 8, 4, 128)
        .transpose(0, 2, 1, 3)
        .reshape(XTR, 128)
    )
    mesh = plsc.VectorSubcoreMesh(core_axis_name="c", subcore_axis_name="s")
    k = pl.kernel(
        _sc_dense_kernel,
        out_type=jax.ShapeDtypeStruct((DTR, 128), jnp.float32),
        mesh=mesh,
        compiler_params=pltpu.CompilerParams(use_tc_tiling_on_sc=False),
        scratch_types=[
            pltpu.VMEM((CTR, 128), jnp.float32),
            pltpu.VMEM((CTR, 128), jnp.float32),
            pltpu.VMEM((CPT, 16), jnp.int32),
            pltpu.VMEM((CTR,), jnp.int32),
            pltpu.SemaphoreType.DMA,
        ],
    )
    out2 = k(x2, params)
    return (
        out2.reshape(B, M // 8, 4, 8, 128)
        .transpose(0, 1, 3, 2, 4)
        .reshape(B, M, F)
    )


def _dmask_body(counts_ref, dmask_ref):
    col = jax.lax.broadcasted_iota(jnp.int32, (B, M), 1)
    dmask_ref[...] = col < counts_ref[...]


def _tc_dmask(counts):
    return pl.pallas_call(
        _dmask_body,
        out_shape=[jax.ShapeDtypeStruct((B, M), jnp.bool_)],
    )(counts.reshape(B, 1))[0]


def kernel(x, batch_idx):
    counts = jnp.sum(
        batch_idx[None, :] == jnp.arange(B, dtype=jnp.int32)[:, None],
        axis=1,
        dtype=jnp.int32,
    )
    ptr_b = jnp.concatenate(
        [jnp.zeros((1,), jnp.int32), jnp.cumsum(counts)[: B - 1]]
    )

    dmask = _tc_dmask(counts)
    mask = jnp.broadcast_to(dmask[:, None, None, :], (B, 1, M, M))

    r = jnp.arange(NCHUNK, dtype=jnp.int32)
    c = (r % CPT) * NW + r // CPT
    cb = c // CPB
    loc0 = (c % CPB) * CR
    srcbase = ptr_b[cb] + loc0
    vlen = jnp.clip(counts[cb] - loc0, 0, CR)
    dchunk = cb * (M // 8 * TRG) + (c % CPB) * CTR
    params = (
        jnp.zeros((NCHUNK, 16), jnp.int32)
        .at[:, 0].set(srcbase)
        .at[:, 1].set(vlen)
        .at[:, 2].set(dchunk)
    )

    dense = _sc_dense(x, params)
    return dense, mask


# speedup vs baseline: 1.6292x; 1.6292x over previous
import jax
import jax.numpy as jnp
from jax import lax
from jax.experimental import pallas as pl
from jax.experimental.pallas import tpu as pltpu
from jax.experimental.pallas import tpu_sc as plsc

B = 16
M = 2048
F = 512
N = 16384

NC = 2
NS = 16
NW = NC * NS
TRG = 32
XTR = N // 8 * 4 * 8
DTR = B * M // 8 * 4 * 8

GC = 4
CR = GC * 8
CTR = GC * TRG
NCHUNK = B * M // CR
CPB = M // CR
CPT = NCHUNK // NW


def _iota16():
    return lax.broadcasted_iota(jnp.int32, (16,), 0)


def _build_idx(idxref, srcbase):
    for t0 in range(0, CTR, 16):
        tv = _iota16() + t0
        j = (tv >> 3) & 3
        sr = srcbase + ((tv >> 5) << 3) + (tv & 7)
        src = jnp.clip(sr, 0, N - 1)
        idxref[pl.ds(t0, 16)] = ((src >> 3) << 5) + (j << 3) + (src & 7)


def _sc_dense_kernel(x_hbm, params_hbm, out_hbm,
                     gbufA, gbufB, zbuf, pbuf, idxA, idxB,
                     sem_g0, sem_g1, sem_o0, sem_o1, sem_z):
    wid = lax.axis_index("s") * NC + lax.axis_index("c")

    pltpu.sync_copy(params_hbm.at[pl.ds(wid * CPT, CPT)], pbuf)

    @pl.loop(0, CTR)
    def _(i):
        @pl.loop(0, 128, step=16)
        def _(j):
            zbuf[i, pl.ds(j, 16)] = jnp.zeros((16,), jnp.float32)

    bufs = ((gbufA, idxA, sem_g0, sem_o0), (gbufB, idxB, sem_g1, sem_o1))

    def step(i, carry):
        d0, d1 = carry

        @pl.when(i < CPT)
        def _():
            row = pbuf[i]
            srcbase = row[0]
            vlen = row[1]
            dchunk = row[2]

            @pl.when(vlen == 0)
            def _():
                pltpu.async_copy(zbuf, out_hbm.at[pl.ds(dchunk, CTR)], sem_z)

            @pl.when(vlen > 0)
            def _():
                for p, (gbuf, idx, sem_g, sem_o) in enumerate(bufs):
                    @pl.when(i % 2 == p)
                    def _(gbuf=gbuf, idx=idx, sem_g=sem_g, sem_o=sem_o,
                          dp=(d0, d1)[p]):
                        @pl.when(dp >= 0)
                        def _():
                            pltpu.make_async_copy(
                                gbuf, out_hbm.at[pl.ds(dp, CTR)], sem_o
                            ).wait()

                        _build_idx(idx, srcbase)
                        pltpu.async_copy(x_hbm.at[idx], gbuf, sem_g)

        prow = pbuf[jnp.maximum(i - 1, 0)]
        pvalid = (i >= 1) & (prow[1] > 0)

        @pl.when(pvalid)
        def _():
            vlen = prow[1]
            for p, (gbuf, idx, sem_g, sem_o) in enumerate(bufs):
                @pl.when((i - 1) % 2 == p)
                def _(gbuf=gbuf, idx=idx, sem_g=sem_g, sem_o=sem_o):
                    pltpu.make_async_copy(x_hbm.at[idx], gbuf, sem_g).wait()

                    @pl.when(vlen < CR)
                    def _():
                        def fix(dr, _):
                            base_tr = ((dr >> 3) << 5) + (dr & 7)
                            for j in range(4):
                                @pl.loop(0, 128, step=16)
                                def _(c, j=j):
                                    gbuf[base_tr + (j << 3), pl.ds(c, 16)] = (
                                        jnp.zeros((16,), jnp.float32)
                                    )
                            return 0

                        lax.fori_loop(vlen, CR, fix, 0)

                    pltpu.async_copy(
                        gbuf, out_hbm.at[pl.ds(prow[2], CTR)], sem_o
                    )

        pp = (i - 1) & 1
        d0 = jnp.where(pvalid & (pp == 0), prow[2], d0)
        d1 = jnp.where(pvalid & (pp == 1), prow[2], d1)
        return d0, d1

    d0, d1 = lax.fori_loop(
        0, CPT + 1, step, (jnp.int32(-1), jnp.int32(-1))
    )

    for p, (gbuf, _idx, _sg, sem_o) in enumerate(bufs):
        dp = (d0, d1)[p]

        @pl.when(dp >= 0)
        def _(gbuf=gbuf, dp=dp, sem_o=sem_o):
            pltpu.make_async_copy(
                gbuf, out_hbm.at[pl.ds(dp, CTR)], sem_o
            ).wait()

    def drain(i, _):
        row = pbuf[i]

        @pl.when(row[1] == 0)
        def _():
            pltpu.make_async_copy(
                zbuf, out_hbm.at[pl.ds(row[2], CTR)], sem_z
            ).wait()

        return 0

    lax.fori_loop(0, CPT, drain, 0)


def _sc_dense(x, params):
    x2 = (
        x.reshape(N // 8, 8, 4, 128)
        .transpose(0, 2, 1, 3)
        .reshape(XTR, 128)
    )
    mesh = plsc.VectorSubcoreMesh(core_axis_name="c", subcore_axis_name="s")
    k = pl.kernel(
        _sc_dense_kernel,
        out_type=jax.ShapeDtypeStruct((DTR, 128), jnp.float32),
        mesh=mesh,
        compiler_params=pltpu.CompilerParams(use_tc_tiling_on_sc=False),
        scratch_types=[
            pltpu.VMEM((CTR, 128), jnp.float32),
            pltpu.VMEM((CTR, 128), jnp.float32),
            pltpu.VMEM((CTR, 128), jnp.float32),
            pltpu.VMEM((CPT, 16), jnp.int32),
            pltpu.VMEM((CTR,), jnp.int32),
            pltpu.VMEM((CTR,), jnp.int32),
            pltpu.SemaphoreType.DMA,
            pltpu.SemaphoreType.DMA,
            pltpu.SemaphoreType.DMA,
            pltpu.SemaphoreType.DMA,
            pltpu.SemaphoreType.DMA,
        ],
    )
    out2 = k(x2, params)
    return (
        out2.reshape(B, M // 8, 4, 8, 128)
        .transpose(0, 1, 3, 2, 4)
        .reshape(B, M, F)
    )


def _dmask_body(counts_ref, dmask_ref):
    col = jax.lax.broadcasted_iota(jnp.int32, (B, M), 1)
    dmask_ref[...] = col < counts_ref[...]


def _tc_dmask(counts):
    return pl.pallas_call(
        _dmask_body,
        out_shape=[jax.ShapeDtypeStruct((B, M), jnp.bool_)],
    )(counts.reshape(B, 1))[0]


def kernel(x, batch_idx):
    counts = jnp.sum(
        batch_idx[None, :] == jnp.arange(B, dtype=jnp.int32)[:, None],
        axis=1,
        dtype=jnp.int32,
    )
    ptr_b = jnp.concatenate(
        [jnp.zeros((1,), jnp.int32), jnp.cumsum(counts)[: B - 1]]
    )

    dmask = _tc_dmask(counts)
    mask = jnp.broadcast_to(dmask[:, None, None, :], (B, 1, M, M))

    loc0 = jnp.arange(CPB, dtype=jnp.int32)[None, :] * CR
    srcbase = ptr_b[:, None] + loc0
    vlen = jnp.clip(counts[:, None] - loc0, 0, CR)
    dchunk = (
        jnp.arange(B, dtype=jnp.int32)[:, None] * (M // 8 * TRG)
        + jnp.arange(CPB, dtype=jnp.int32)[None, :] * CTR
    )
    table = jnp.stack(
        [srcbase.reshape(-1), vlen.reshape(-1), dchunk.reshape(-1)], axis=-1
    )
    table = jnp.pad(table, ((0, 0), (0, 13)))
    params = (
        table.reshape(CPT, NW, 16).transpose(1, 0, 2).reshape(NCHUNK, 16)
    )

    dense = _sc_dense(x, params)
    return dense, mask
